# baseline (speedup 1.0000x reference)
import jax
import jax.numpy as jnp
from jax import lax
from jax.experimental import pallas as pl
from jax.experimental.pallas import tpu as pltpu
from jax.experimental.pallas import tpu_sc as plsc

_N = 10000
_E = 320000
_D = 128
_H = 256
_C = 32
_G = 64

_NP = 10240
_EP = 327680
_CHUNK = 128
_NSUB = 16
_NCORE = 2
_HALF = 128

_ROWS_PER_TILE = _NP // _NSUB
_SCAT_CHUNKS = _EP // (_NSUB * _CHUNK)
_EDGES_PER_TILE = _EP // _NSUB
_DEG_EDGES = _EP // (_NCORE * _NSUB)
_HROWS = _NP // 128
_DUMMY = _N

_NPASS = 2
_RANGE = _NP // _NPASS
_ACC_ROWS = _RANGE
_DRAIN_ROWS = _RANGE // _NSUB
_ZSLICES = _ACC_ROWS // _CHUNK

_NW = _NCORE * _NSUB
_WEDGES = _EP // _NW
_WCAP = _WEDGES // _CHUNK
_PSIZE = _NW * _NPASS * _WEDGES
_PROWS = _PSIZE // _CHUNK
_PSIZE_D = _PSIZE + _NW * _CHUNK

_BLK = 512
_PBLK = 2048


def _sc_mesh():
    return plsc.VectorSubcoreMesh(core_axis_name="c", subcore_axis_name="s",
                                  num_cores=_NCORE)


_SC_PARAMS = pltpu.CompilerParams(needs_layout_passes=False)


def _part_body(src1, dst1, psrc, pdst, h0, h1,
               in_s, in_d, ob_s0, ob_d0, ob_s1, ob_d1, row_v,
               hist_v, idrow_v, hacc):
    cid = lax.axis_index("c")
    sid = lax.axis_index("s")
    wid = sid * _NCORE + cid
    base = wid * _WEDGES
    pltpu.sync_copy(src1.at[pl.ds(base, _WEDGES)], in_s)
    pltpu.sync_copy(dst1.at[pl.ds(base, _WEDGES)], in_d)

    iot = lax.iota(jnp.int32, 16)

    @pl.loop(0, _HROWS)
    def _(r):
        for c in range(8):
            hist_v[r, pl.ds(c * 16, 16)] = jnp.zeros((16,), jnp.float32)

    @pl.when(sid < _HROWS // 8)
    def _():
        pltpu.sync_copy(hist_v.at[pl.ds(0, 8)], hacc.at[pl.ds(sid * 8, 8)])

    for r in range(_HROWS // 16):
        idrow_v[0, pl.ds(r * 16, 16)] = lax.iota(jnp.int32, 16) + r * 16

    def group(g, carry):
        lo0, lo1 = carry
        s = in_s[pl.ds(g * 16, 16)]
        d = in_d[pl.ds(g * 16, 16)]
        m0 = d < _RANGE
        m1 = jnp.logical_and(d >= _RANGE, d < _NP)
        i0 = m0.astype(jnp.int32)
        i1 = m1.astype(jnp.int32)
        r0 = plsc.cumsum(i0) - i0
        r1 = plsc.cumsum(i1) - i1
        dest0 = jnp.where(m0, r0 + lo0, 0)
        dest1 = jnp.where(m1, r1 + lo1, 0)
        plsc.store_scatter(ob_s0, [dest0], s, mask=m0)
        plsc.store_scatter(ob_d0, [dest0], d, mask=m0)
        plsc.store_scatter(ob_s1, [dest1], s, mask=m1)
        plsc.store_scatter(ob_d1, [dest1], d - _RANGE, mask=m1)
        m01 = jnp.logical_or(m0, m1)
        dc = jnp.where(m01, d, 0)
        plsc.addupdate_scatter(
            hist_v, [lax.shift_right_logical(dc, 7),
                     lax.bitwise_and(dc, 127)],
            jnp.ones((16,), jnp.float32),
            mask=m01)
        return lo0 + jnp.sum(i0), lo1 + jnp.sum(i1)

    lo0, lo1 = lax.fori_loop(0, _WEDGES // 16, group, (0, 0))

    plsc.subcore_barrier()
    pltpu.sync_copy(hist_v, hacc.at[idrow_v.at[0]], add=True)
    plsc.subcore_barrier()

    @pl.when(jnp.logical_and(cid == 0, sid < _HROWS // 8))
    def _():
        pltpu.sync_copy(hacc.at[pl.ds(sid * 8, 8)],
                        h0.at[pl.ds(sid * 8, 8)])

    @pl.when(jnp.logical_and(cid == 1, sid < _HROWS // 8))
    def _():
        pltpu.sync_copy(hacc.at[pl.ds(sid * 8, 8)],
                        h1.at[pl.ds(sid * 8, 8)])

    zsrc = (_NP - 16) + iot

    def pad(ob_s, ob_d, lo):
        tgt = lax.div(lo + _CHUNK - 1, _CHUNK) * _CHUNK
        for r in range(_CHUNK // 16):
            idx = lo + r * 16 + iot
            m = idx < tgt
            idxc = jnp.where(m, idx, 0)
            plsc.store_scatter(ob_s, [idxc], zsrc, mask=m)
            plsc.store_scatter(ob_d, [idxc], iot, mask=m)
        return lax.div(tgt, _CHUNK)

    nch0 = pad(ob_s0, ob_d0, lo0)
    nch1 = pad(ob_s1, ob_d1, lo1)

    pltpu.sync_copy(ob_s0, psrc.at[pl.ds((wid * _NPASS) * _WEDGES, _WEDGES)])
    pltpu.sync_copy(ob_d0, pdst.at[pl.ds((wid * _NPASS) * _WEDGES, _WEDGES)])
    pltpu.sync_copy(ob_s1,
                    psrc.at[pl.ds((wid * _NPASS + 1) * _WEDGES, _WEDGES)])
    pltpu.sync_copy(ob_d1,
                    pdst.at[pl.ds((wid * _NPASS + 1) * _WEDGES, _WEDGES)])

    row_v[pl.ds(0, 16)] = (nch0 * (iot == 0).astype(jnp.int32)
                           + nch1 * (iot == 1).astype(jnp.int32))
    for k in range(1, 128 // 16):
        row_v[pl.ds(k * 16, 16)] = jnp.zeros((16,), jnp.int32)
    pltpu.sync_copy(row_v, pdst.at[pl.ds(_PSIZE + wid * _CHUNK, _CHUNK)])


def _part_call(src1, dst1):
    f = pl.kernel(
        _part_body,
        out_type=[jax.ShapeDtypeStruct((_PSIZE,), jnp.int32),
                  jax.ShapeDtypeStruct((_PSIZE_D,), jnp.int32),
                  jax.ShapeDtypeStruct((_HROWS, 128), jnp.float32),
                  jax.ShapeDtypeStruct((_HROWS, 128), jnp.float32)],
        mesh=_sc_mesh(),
        scratch_types=[
            pltpu.VMEM((_WEDGES,), jnp.int32),
            pltpu.VMEM((_WEDGES,), jnp.int32),
            pltpu.VMEM((_WEDGES,), jnp.int32),
            pltpu.VMEM((_WEDGES,), jnp.int32),
            pltpu.VMEM((_WEDGES,), jnp.int32),
            pltpu.VMEM((_WEDGES,), jnp.int32),
            pltpu.VMEM((128,), jnp.int32),
            pltpu.VMEM((_HROWS, 128), jnp.float32),
            pltpu.VMEM((1, _HROWS), jnp.int32),
            pltpu.VMEM_SHARED((_HROWS, 128), jnp.float32),
        ],
        compiler_params=_SC_PARAMS,
    )
    return f(src1, dst1)


def _scat_body(tt0, tt1, psrc, pdst, s0, s1,
               isrc_v, idst_v, rows_v, acc, *sems):
    gsems = sems[:3]
    ssems = sems[3:]
    cid = lax.axis_index("c")
    sid = lax.axis_index("s")

    iot = lax.iota(jnp.int32, 16)
    wa = 2 * sid
    wb = 2 * sid + 1
    pltpu.sync_copy(pdst.at[pl.ds(_PROWS + wa, 2)],
                    idst_v.at[pl.ds(0, 2)])

    def chunk_count(i, p):
        return jnp.sum(idst_v[i, pl.ds(0, 16)]
                       * (iot == p).astype(jnp.int32))

    counts = [[chunk_count(i, p) for i in (0, 1)] for p in (0, 1)]

    def one_pass(p, tt, s_out):
        ra = (wa * _NPASS + p) * _WCAP
        rb = (wb * _NPASS + p) * _WCAP
        pltpu.sync_copy(psrc.at[pl.ds(ra, _WCAP)],
                        isrc_v.at[pl.ds(0, _WCAP)])
        pltpu.sync_copy(psrc.at[pl.ds(rb, _WCAP)],
                        isrc_v.at[pl.ds(_WCAP, _WCAP)])
        pltpu.sync_copy(pdst.at[pl.ds(ra, _WCAP)],
                        idst_v.at[pl.ds(0, _WCAP)])
        pltpu.sync_copy(pdst.at[pl.ds(rb, _WCAP)],
                        idst_v.at[pl.ds(_WCAP, _WCAP)])
        ncha = counts[p][0]
        ntot = ncha + counts[p][1]

        def rowof(jj):
            return jnp.where(jj < ncha, jj, _WCAP + jj - ncha)

        @pl.loop(0, _CHUNK)
        def _(r):
            for c in range(_HALF // 16):
                rows_v[0, r, pl.ds(c * 16, 16)] = jnp.zeros(
                    (16,), jnp.float32)

        for m in range((_ZSLICES + _NSUB - 1) // _NSUB):
            k = m * _NSUB

            @pl.when(sid + k < _ZSLICES)
            def _():
                pltpu.sync_copy(rows_v.at[0],
                                acc.at[pl.ds((sid + k) * _CHUNK, _CHUNK)])
        plsc.subcore_barrier()

        @pl.when(ntot > 0)
        def _():
            pltpu.make_async_copy(tt.at[isrc_v.at[rowof(0)]], rows_v.at[0],
                                  gsems[0]).start()

        @pl.when(ntot > 1)
        def _():
            pltpu.make_async_copy(tt.at[isrc_v.at[rowof(1)]], rows_v.at[1],
                                  gsems[1]).start()

        @pl.loop(0, ntot, step=3)
        def _(j):
            for t in range(3):
                jj = j + t

                @pl.when(jj < ntot)
                def _():
                    r = rowof(jj)
                    pltpu.make_async_copy(tt.at[isrc_v.at[r]],
                                          rows_v.at[t], gsems[t]).wait()
                    pltpu.async_copy(rows_v.at[t], acc.at[idst_v.at[r]],
                                     ssems[t], add=True)
                    nxt = jj + 2
                    t2 = (t + 2) % 3

                    @pl.when(nxt < ntot)
                    def _():
                        @pl.when(nxt >= 3)
                        def _():
                            pltpu.make_async_copy(
                                rows_v.at[t2], acc.at[idst_v.at[0]],
                                ssems[t2]).wait()

                        pltpu.make_async_copy(tt.at[isrc_v.at[rowof(nxt)]],
                                              rows_v.at[t2],
                                              gsems[t2]).start()

        for t in range(3):
            @pl.when(t < ntot)
            def _():
                pltpu.make_async_copy(rows_v.at[t], acc.at[idst_v.at[0]],
                                      ssems[t]).wait()

        plsc.subcore_barrier()
        pltpu.sync_copy(acc.at[pl.ds(sid * _DRAIN_ROWS, _DRAIN_ROWS)],
                        s_out.at[pl.ds(p * _RANGE + sid * _DRAIN_ROWS,
                                       _DRAIN_ROWS)])
        plsc.subcore_barrier()

    @pl.when(cid == 0)
    def _():
        one_pass(0, tt0, s0)
        one_pass(1, tt0, s0)

    @pl.when(cid == 1)
    def _():
        one_pass(0, tt1, s1)
        one_pass(1, tt1, s1)


def _scat_call(tts, psrc, pdst):
    f = pl.kernel(
        _scat_body,
        out_type=[jax.ShapeDtypeStruct((_NP, _HALF), jnp.float32),
                  jax.ShapeDtypeStruct((_NP, _HALF), jnp.float32)],
        mesh=_sc_mesh(),
        scratch_types=[
            pltpu.VMEM((2 * _WCAP, _CHUNK), jnp.int32),
            pltpu.VMEM((2 * _WCAP, _CHUNK), jnp.int32),
            pltpu.VMEM((3, _CHUNK, _HALF), jnp.float32),
            pltpu.VMEM_SHARED((_ACC_ROWS, _HALF), jnp.float32),
        ] + [pltpu.SemaphoreType.DMA] * 6,
        compiler_params=_SC_PARAMS,
    )
    return f(*tts, psrc, pdst)


def _k1_body(x_ref, w_ref, dg_ref, t0_ref, t1_ref):
    dinv = lax.rsqrt(dg_ref[...] + 1.0)
    t = jnp.dot(x_ref[...], w_ref[...], preferred_element_type=jnp.float32)
    tt = t * dinv
    t0_ref[...] = tt[:, :_HALF]
    t1_ref[...] = tt[:, _HALF:]


def _k1_call(x, w, dg):
    return pl.pallas_call(
        _k1_body,
        grid=(_NP // _BLK,),
        in_specs=[
            pl.BlockSpec((_BLK, _D), lambda i: (i, 0)),
            pl.BlockSpec((_D, _H), lambda i: (0, 0)),
            pl.BlockSpec((_BLK, 1), lambda i: (i, 0)),
        ],
        out_specs=[
            pl.BlockSpec((_BLK, _HALF), lambda i: (i, 0)),
            pl.BlockSpec((_BLK, _HALF), lambda i: (i, 0)),
        ],
        out_shape=[jax.ShapeDtypeStruct((_NP, _HALF), jnp.float32),
                   jax.ShapeDtypeStruct((_NP, _HALF), jnp.float32)],
    )(x, w, dg)


def _layer_body(s0_ref, s1_ref, t0_ref, t1_ref, dg_ref, b_ref, w_ref,
                o0_ref, o1_ref):
    i = pl.program_id(0)
    dinv = lax.rsqrt(dg_ref[...] + 1.0)
    u0 = (s0_ref[...] + t0_ref[...]) * dinv
    u1 = (s1_ref[...] + t1_ref[...]) * dinv
    a0 = jnp.maximum(u0 + b_ref[:, :_HALF], 0.0)
    a1 = jnp.maximum(u1 + b_ref[:, _HALF:], 0.0)
    t = (jnp.dot(a0, w_ref[:_HALF, :], preferred_element_type=jnp.float32)
         + jnp.dot(a1, w_ref[_HALF:, :], preferred_element_type=jnp.float32))
    rid = i * _BLK + lax.broadcasted_iota(jnp.int32, (_BLK, 1), 0)
    tt = t * jnp.where(rid < _N, dinv, 0.0)
    o0_ref[...] = tt[:, :_HALF]
    o1_ref[...] = tt[:, _HALF:]


def _layer_call(ss, tts, dg, b, w):
    return pl.pallas_call(
        _layer_body,
        grid=(_NP // _BLK,),
        in_specs=[
            pl.BlockSpec((_BLK, _HALF), lambda i: (i, 0)),
            pl.BlockSpec((_BLK, _HALF), lambda i: (i, 0)),
            pl.BlockSpec((_BLK, _HALF), lambda i: (i, 0)),
            pl.BlockSpec((_BLK, _HALF), lambda i: (i, 0)),
            pl.BlockSpec((_BLK, 1), lambda i: (i, 0)),
            pl.BlockSpec((1, _H), lambda i: (0, 0)),
            pl.BlockSpec((_H, _H), lambda i: (0, 0)),
        ],
        out_specs=[
            pl.BlockSpec((_BLK, _HALF), lambda i: (i, 0)),
            pl.BlockSpec((_BLK, _HALF), lambda i: (i, 0)),
        ],
        out_shape=[jax.ShapeDtypeStruct((_NP, _HALF), jnp.float32),
                   jax.ShapeDtypeStruct((_NP, _HALF), jnp.float32)],
    )(*ss, *tts, dg, b, w)


def _head_body(s0_ref, s1_ref, t0_ref, t1_ref, dg_ref, b3_ref, batch_ref,
               w1_ref, b1_ref, w2_ref, b2_ref, o_ref, acc):
    i = pl.program_id(0)
    dinv = lax.rsqrt(dg_ref[...] + 1.0)
    u0 = (s0_ref[...] + t0_ref[...]) * dinv
    u1 = (s1_ref[...] + t1_ref[...]) * dinv
    h0 = jnp.maximum(u0 + b3_ref[:, :_HALF], 0.0)
    h1 = jnp.maximum(u1 + b3_ref[:, _HALF:], 0.0)

    bvals = batch_ref[...].reshape(1, _PBLK)
    gids = lax.broadcasted_iota(jnp.int32, (_G, _PBLK), 0)
    mask = (bvals == gids).astype(jnp.float32)

    @pl.when(i == 0)
    def _():
        acc[...] = jnp.zeros((_G, _H), jnp.float32)

    acc[:, :_HALF] += jnp.dot(mask, h0, preferred_element_type=jnp.float32)
    acc[:, _HALF:] += jnp.dot(mask, h1, preferred_element_type=jnp.float32)

    @pl.when(i == pl.num_programs(0) - 1)
    def _():
        p = acc[...]
        z1 = jnp.maximum(
            jnp.dot(p, w1_ref[...], preferred_element_type=jnp.float32)
            + b1_ref[...], 0.0)
        z = (jnp.dot(z1, w2_ref[...], preferred_element_type=jnp.float32)
             + b2_ref[...])
        m = jnp.max(z, axis=1, keepdims=True)
        e = jnp.exp(z - m)
        lse = jnp.log(jnp.sum(e, axis=1, keepdims=True)) + m
        o_ref[...] = z - lse


def _head_call(ss, tts, dg, b3, batch3d, w1, b1, w2, b2):
    return pl.pallas_call(
        _head_body,
        grid=(_NP // _PBLK,),
        in_specs=[
            pl.BlockSpec((_PBLK, _HALF), lambda i: (i, 0)),
            pl.BlockSpec((_PBLK, _HALF), lambda i: (i, 0)),
            pl.BlockSpec((_PBLK, _HALF), lambda i: (i, 0)),
            pl.BlockSpec((_PBLK, _HALF), lambda i: (i, 0)),
            pl.BlockSpec((_PBLK, 1), lambda i: (i, 0)),
            pl.BlockSpec((1, _H), lambda i: (0, 0)),
            pl.BlockSpec((1, 1, _PBLK), lambda i: (i, 0, 0)),
            pl.BlockSpec((_H, _H), lambda i: (0, 0)),
            pl.BlockSpec((1, _H), lambda i: (0, 0)),
            pl.BlockSpec((_H, _C), lambda i: (0, 0)),
            pl.BlockSpec((1, _C), lambda i: (0, 0)),
        ],
        out_specs=pl.BlockSpec((_G, _C), lambda i: (0, 0)),
        out_shape=jax.ShapeDtypeStruct((_G, _C), jnp.float32),
        scratch_shapes=[pltpu.VMEM((_G, _H), jnp.float32)],
    )(*ss, *tts, dg, b3, batch3d, w1, b1, w2, b2)


def kernel(x, edge_index, batch, W1, b1, W2, b2, W3, b3,
           lin1_W, lin1_b, lin2_W, lin2_b):
    x_pad = jnp.pad(x, ((0, _NP - _N), (0, 0)))
    src = jnp.concatenate(
        [edge_index[0], jnp.zeros((_EP - _E,), jnp.int32)])
    dst_sent = jnp.concatenate(
        [edge_index[1], jnp.full((_EP - _E,), 1 << 20, jnp.int32)])

    batch3d = jnp.pad(batch, (0, _NP - _N), constant_values=_G).reshape(
        _NP // _PBLK, 1, _PBLK)
    b1r = b1.reshape(1, _H)
    b2r = b2.reshape(1, _H)
    b3r = b3.reshape(1, _H)
    l1br = lin1_b.reshape(1, _H)
    l2br = lin2_b.reshape(1, _C)

    ps1, pd1, h0, h1 = _part_call(src, dst_sent)
    psrc = ps1.reshape(_PROWS, _CHUNK)
    pdst = pd1.reshape(_PSIZE_D // _CHUNK, _CHUNK)
    dg = (h0 + h1).reshape(_NP, 1)
    tts = _k1_call(x_pad, W1, dg)
    ss = _scat_call(tts, psrc, pdst)
    tts = _layer_call(ss, tts, dg, b1r, W2)
    ss = _scat_call(tts, psrc, pdst)
    tts = _layer_call(ss, tts, dg, b2r, W3)
    ss = _scat_call(tts, psrc, pdst)
    return _head_call(ss, tts, dg, b3r, batch3d,
                      lin1_W, l1br, lin2_W, l2br)


# speedup vs baseline: 1.0100x; 1.0100x over previous
import jax
import jax.numpy as jnp
from jax import lax
from jax.experimental import pallas as pl
from jax.experimental.pallas import tpu as pltpu
from jax.experimental.pallas import tpu_sc as plsc

_N = 10000
_E = 320000
_D = 128
_H = 256
_C = 32
_G = 64

_NP = 10240
_EP = 327680
_CHUNK = 128
_NSUB = 16
_NCORE = 2
_HALF = 128

_DEG_EDGES = _EP // (_NCORE * _NSUB)
_HROWS = _NP // 128
_DUMMY = _N

_NPASS = 2
_RANGE = _NP // _NPASS
_ACC_ROWS = _RANGE
_DRAIN_ROWS = _RANGE // _NSUB
_ZSLICES = _ACC_ROWS // _CHUNK

_NW = _NCORE * _NSUB
_WEDGES = _EP // _NW
_WCAP = _WEDGES // _CHUNK
_PSIZE = _NW * _NPASS * _WEDGES
_PROWS = _PSIZE // _CHUNK
_PSIZE_D = _PSIZE + _NW * _CHUNK

_BLK = 512
_PBLK = 2048


def _sc_mesh():
    return plsc.VectorSubcoreMesh(core_axis_name="c", subcore_axis_name="s",
                                  num_cores=_NCORE)


_SC_PARAMS = pltpu.CompilerParams(needs_layout_passes=False)


def _deg_body(dst2d, h0, h1, idx_v, hist_v, idrow_v, acc):
    cid = lax.axis_index("c")
    sid = lax.axis_index("s")

    @pl.loop(0, _HROWS)
    def _(r):
        for c in range(8):
            hist_v[r, pl.ds(c * 16, 16)] = jnp.zeros((16,), jnp.float32)

    @pl.when(sid < _HROWS // 8)
    def _():
        pltpu.sync_copy(hist_v.at[pl.ds(0, 8)], acc.at[pl.ds(sid * 8, 8)])

    for r in range(_HROWS // 16):
        idrow_v[0, pl.ds(r * 16, 16)] = lax.iota(jnp.int32, 16) + r * 16

    wid = sid * _NCORE + cid
    rows = _DEG_EDGES // _CHUNK
    pltpu.sync_copy(dst2d.at[pl.ds(wid * rows, rows)], idx_v)

    @pl.loop(0, rows)
    def _(r):
        for c in range(_CHUNK // 16):
            v = idx_v[r, pl.ds(c * 16, 16)]
            plsc.addupdate_scatter(
                hist_v, [lax.shift_right_logical(v, 7),
                         lax.bitwise_and(v, 127)],
                jnp.ones((16,), jnp.float32))

    plsc.subcore_barrier()
    pltpu.sync_copy(hist_v, acc.at[idrow_v.at[0]], add=True)
    plsc.subcore_barrier()

    @pl.when(jnp.logical_and(cid == 0, sid < _HROWS // 8))
    def _():
        pltpu.sync_copy(acc.at[pl.ds(sid * 8, 8)],
                        h0.at[pl.ds(sid * 8, 8)])

    @pl.when(jnp.logical_and(cid == 1, sid < _HROWS // 8))
    def _():
        pltpu.sync_copy(acc.at[pl.ds(sid * 8, 8)],
                        h1.at[pl.ds(sid * 8, 8)])


def _deg_call(dst2d):
    f = pl.kernel(
        _deg_body,
        out_type=[jax.ShapeDtypeStruct((_HROWS, 128), jnp.float32),
                  jax.ShapeDtypeStruct((_HROWS, 128), jnp.float32)],
        mesh=_sc_mesh(),
        scratch_types=[
            pltpu.VMEM((_DEG_EDGES // _CHUNK, _CHUNK), jnp.int32),
            pltpu.VMEM((_HROWS, 128), jnp.float32),
            pltpu.VMEM((1, _HROWS), jnp.int32),
            pltpu.VMEM_SHARED((_HROWS, 128), jnp.float32),
        ],
        compiler_params=_SC_PARAMS,
    )
    return f(dst2d)


def _part_body(src1, dst1, psrc, pdst,
               in_s, in_d, ob_s0, ob_d0, ob_s1, ob_d1, row_v):
    cid = lax.axis_index("c")
    sid = lax.axis_index("s")
    wid = sid * _NCORE + cid
    base = wid * _WEDGES
    pltpu.sync_copy(src1.at[pl.ds(base, _WEDGES)], in_s)
    pltpu.sync_copy(dst1.at[pl.ds(base, _WEDGES)], in_d)

    iot = lax.iota(jnp.int32, 16)

    def group(g, carry):
        lo0, lo1 = carry
        s = in_s[pl.ds(g * 16, 16)]
        d = in_d[pl.ds(g * 16, 16)]
        m0 = d < _RANGE
        m1 = jnp.logical_and(d >= _RANGE, d < _NP)
        i0 = m0.astype(jnp.int32)
        i1 = m1.astype(jnp.int32)
        r0 = plsc.cumsum(i0) - i0
        r1 = plsc.cumsum(i1) - i1
        dest0 = jnp.where(m0, r0 + lo0, 0)
        dest1 = jnp.where(m1, r1 + lo1, 0)
        plsc.store_scatter(ob_s0, [dest0], s, mask=m0)
        plsc.store_scatter(ob_d0, [dest0], d, mask=m0)
        plsc.store_scatter(ob_s1, [dest1], s, mask=m1)
        plsc.store_scatter(ob_d1, [dest1], d - _RANGE, mask=m1)
        return lo0 + jnp.sum(i0), lo1 + jnp.sum(i1)

    lo0, lo1 = lax.fori_loop(0, _WEDGES // 16, group, (0, 0))

    zsrc = (_NP - 16) + iot

    def pad(ob_s, ob_d, lo):
        tgt = lax.div(lo + _CHUNK - 1, _CHUNK) * _CHUNK
        for r in range(_CHUNK // 16):
            idx = lo + r * 16 + iot
            m = idx < tgt
            idxc = jnp.where(m, idx, 0)
            plsc.store_scatter(ob_s, [idxc], zsrc, mask=m)
            plsc.store_scatter(ob_d, [idxc], iot, mask=m)
        return lax.div(tgt, _CHUNK)

    nch0 = pad(ob_s0, ob_d0, lo0)
    nch1 = pad(ob_s1, ob_d1, lo1)

    pltpu.sync_copy(ob_s0, psrc.at[pl.ds((wid * _NPASS) * _WEDGES, _WEDGES)])
    pltpu.sync_copy(ob_d0, pdst.at[pl.ds((wid * _NPASS) * _WEDGES, _WEDGES)])
    pltpu.sync_copy(ob_s1,
                    psrc.at[pl.ds((wid * _NPASS + 1) * _WEDGES, _WEDGES)])
    pltpu.sync_copy(ob_d1,
                    pdst.at[pl.ds((wid * _NPASS + 1) * _WEDGES, _WEDGES)])

    row_v[pl.ds(0, 16)] = (nch0 * (iot == 0).astype(jnp.int32)
                           + nch1 * (iot == 1).astype(jnp.int32))
    for k in range(1, 128 // 16):
        row_v[pl.ds(k * 16, 16)] = jnp.zeros((16,), jnp.int32)
    pltpu.sync_copy(row_v, pdst.at[pl.ds(_PSIZE + wid * _CHUNK, _CHUNK)])


def _part_call(src1, dst1):
    f = pl.kernel(
        _part_body,
        out_type=[jax.ShapeDtypeStruct((_PSIZE,), jnp.int32),
                  jax.ShapeDtypeStruct((_PSIZE_D,), jnp.int32)],
        mesh=_sc_mesh(),
        scratch_types=[
            pltpu.VMEM((_WEDGES,), jnp.int32),
            pltpu.VMEM((_WEDGES,), jnp.int32),
            pltpu.VMEM((_WEDGES,), jnp.int32),
            pltpu.VMEM((_WEDGES,), jnp.int32),
            pltpu.VMEM((_WEDGES,), jnp.int32),
            pltpu.VMEM((_WEDGES,), jnp.int32),
            pltpu.VMEM((128,), jnp.int32),
        ],
        compiler_params=_SC_PARAMS,
    )
    return f(src1, dst1)


def _scat_body(tt0, tt1, psrc, pdst, s0, s1,
               isrc_v, idst_v, rows_v, acc, *sems):
    gsems = sems[:3]
    ssems = sems[3:]
    cid = lax.axis_index("c")
    sid = lax.axis_index("s")

    iot = lax.iota(jnp.int32, 16)
    wa = 2 * sid
    wb = 2 * sid + 1
    pltpu.sync_copy(pdst.at[pl.ds(_PROWS + wa, 2)],
                    idst_v.at[pl.ds(0, 2)])

    def chunk_count(i, p):
        return jnp.sum(idst_v[i, pl.ds(0, 16)]
                       * (iot == p).astype(jnp.int32))

    counts = [[chunk_count(i, p) for i in (0, 1)] for p in (0, 1)]

    def one_pass(p, tt, s_out):
        ra = (wa * _NPASS + p) * _WCAP
        rb = (wb * _NPASS + p) * _WCAP
        pltpu.sync_copy(psrc.at[pl.ds(ra, _WCAP)],
                        isrc_v.at[pl.ds(0, _WCAP)])
        pltpu.sync_copy(psrc.at[pl.ds(rb, _WCAP)],
                        isrc_v.at[pl.ds(_WCAP, _WCAP)])
        pltpu.sync_copy(pdst.at[pl.ds(ra, _WCAP)],
                        idst_v.at[pl.ds(0, _WCAP)])
        pltpu.sync_copy(pdst.at[pl.ds(rb, _WCAP)],
                        idst_v.at[pl.ds(_WCAP, _WCAP)])
        ncha = counts[p][0]
        ntot = ncha + counts[p][1]

        def rowof(jj):
            return jnp.where(jj < ncha, jj, _WCAP + jj - ncha)

        @pl.loop(0, _CHUNK)
        def _(r):
            for c in range(_HALF // 16):
                rows_v[0, r, pl.ds(c * 16, 16)] = jnp.zeros(
                    (16,), jnp.float32)

        for m in range((_ZSLICES + _NSUB - 1) // _NSUB):
            k = m * _NSUB

            @pl.when(sid + k < _ZSLICES)
            def _():
                pltpu.sync_copy(rows_v.at[0],
                                acc.at[pl.ds((sid + k) * _CHUNK, _CHUNK)])
        plsc.subcore_barrier()

        @pl.when(ntot > 0)
        def _():
            pltpu.make_async_copy(tt.at[isrc_v.at[rowof(0)]], rows_v.at[0],
                                  gsems[0]).start()

        @pl.when(ntot > 1)
        def _():
            pltpu.make_async_copy(tt.at[isrc_v.at[rowof(1)]], rows_v.at[1],
                                  gsems[1]).start()

        @pl.loop(0, ntot, step=3)
        def _(j):
            for t in range(3):
                jj = j + t

                @pl.when(jj < ntot)
                def _():
                    r = rowof(jj)
                    pltpu.make_async_copy(tt.at[isrc_v.at[r]],
                                          rows_v.at[t], gsems[t]).wait()
                    pltpu.async_copy(rows_v.at[t], acc.at[idst_v.at[r]],
                                     ssems[t], add=True)
                    nxt = jj + 2
                    t2 = (t + 2) % 3

                    @pl.when(nxt < ntot)
                    def _():
                        @pl.when(nxt >= 3)
                        def _():
                            pltpu.make_async_copy(
                                rows_v.at[t2], acc.at[idst_v.at[0]],
                                ssems[t2]).wait()

                        pltpu.make_async_copy(tt.at[isrc_v.at[rowof(nxt)]],
                                              rows_v.at[t2],
                                              gsems[t2]).start()

        for t in range(3):
            @pl.when(t < ntot)
            def _():
                pltpu.make_async_copy(rows_v.at[t], acc.at[idst_v.at[0]],
                                      ssems[t]).wait()

        plsc.subcore_barrier()
        pltpu.sync_copy(acc.at[pl.ds(sid * _DRAIN_ROWS, _DRAIN_ROWS)],
                        s_out.at[pl.ds(p * _RANGE + sid * _DRAIN_ROWS,
                                       _DRAIN_ROWS)])
        plsc.subcore_barrier()

    @pl.when(cid == 0)
    def _():
        one_pass(0, tt0, s0)
        one_pass(1, tt0, s0)

    @pl.when(cid == 1)
    def _():
        one_pass(0, tt1, s1)
        one_pass(1, tt1, s1)


def _scat_call(tts, psrc, pdst):
    f = pl.kernel(
        _scat_body,
        out_type=[jax.ShapeDtypeStruct((_NP, _HALF), jnp.float32),
                  jax.ShapeDtypeStruct((_NP, _HALF), jnp.float32)],
        mesh=_sc_mesh(),
        scratch_types=[
            pltpu.VMEM((2 * _WCAP, _CHUNK), jnp.int32),
            pltpu.VMEM((2 * _WCAP, _CHUNK), jnp.int32),
            pltpu.VMEM((3, _CHUNK, _HALF), jnp.float32),
            pltpu.VMEM_SHARED((_ACC_ROWS, _HALF), jnp.float32),
        ] + [pltpu.SemaphoreType.DMA] * 6,
        compiler_params=_SC_PARAMS,
    )
    return f(*tts, psrc, pdst)


def _k1_body(x_ref, w_ref, dg_ref, t0_ref, t1_ref):
    dinv = lax.rsqrt(dg_ref[...] + 1.0)
    t = jnp.dot(x_ref[...], w_ref[...], preferred_element_type=jnp.float32)
    tt = t * dinv
    t0_ref[...] = tt[:, :_HALF]
    t1_ref[...] = tt[:, _HALF:]


def _k1_call(x, w, dg):
    return pl.pallas_call(
        _k1_body,
        grid=(_NP // _BLK,),
        in_specs=[
            pl.BlockSpec((_BLK, _D), lambda i: (i, 0)),
            pl.BlockSpec((_D, _H), lambda i: (0, 0)),
            pl.BlockSpec((_BLK, 1), lambda i: (i, 0)),
        ],
        out_specs=[
            pl.BlockSpec((_BLK, _HALF), lambda i: (i, 0)),
            pl.BlockSpec((_BLK, _HALF), lambda i: (i, 0)),
        ],
        out_shape=[jax.ShapeDtypeStruct((_NP, _HALF), jnp.float32),
                   jax.ShapeDtypeStruct((_NP, _HALF), jnp.float32)],
    )(x, w, dg)


def _layer_body(s0_ref, s1_ref, t0_ref, t1_ref, dg_ref, b_ref, w_ref,
                o0_ref, o1_ref):
    i = pl.program_id(0)
    dinv = lax.rsqrt(dg_ref[...] + 1.0)
    u0 = (s0_ref[...] + t0_ref[...]) * dinv
    u1 = (s1_ref[...] + t1_ref[...]) * dinv
    a0 = jnp.maximum(u0 + b_ref[:, :_HALF], 0.0)
    a1 = jnp.maximum(u1 + b_ref[:, _HALF:], 0.0)
    t = (jnp.dot(a0, w_ref[:_HALF, :], preferred_element_type=jnp.float32)
         + jnp.dot(a1, w_ref[_HALF:, :], preferred_element_type=jnp.float32))
    rid = i * _BLK + lax.broadcasted_iota(jnp.int32, (_BLK, 1), 0)
    tt = t * jnp.where(rid < _N, dinv, 0.0)
    o0_ref[...] = tt[:, :_HALF]
    o1_ref[...] = tt[:, _HALF:]


def _layer_call(ss, tts, dg, b, w):
    return pl.pallas_call(
        _layer_body,
        grid=(_NP // _BLK,),
        in_specs=[
            pl.BlockSpec((_BLK, _HALF), lambda i: (i, 0)),
            pl.BlockSpec((_BLK, _HALF), lambda i: (i, 0)),
            pl.BlockSpec((_BLK, _HALF), lambda i: (i, 0)),
            pl.BlockSpec((_BLK, _HALF), lambda i: (i, 0)),
            pl.BlockSpec((_BLK, 1), lambda i: (i, 0)),
            pl.BlockSpec((1, _H), lambda i: (0, 0)),
            pl.BlockSpec((_H, _H), lambda i: (0, 0)),
        ],
        out_specs=[
            pl.BlockSpec((_BLK, _HALF), lambda i: (i, 0)),
            pl.BlockSpec((_BLK, _HALF), lambda i: (i, 0)),
        ],
        out_shape=[jax.ShapeDtypeStruct((_NP, _HALF), jnp.float32),
                   jax.ShapeDtypeStruct((_NP, _HALF), jnp.float32)],
    )(*ss, *tts, dg, b, w)


def _head_body(s0_ref, s1_ref, t0_ref, t1_ref, dg_ref, b3_ref, batch_ref,
               w1_ref, b1_ref, w2_ref, b2_ref, o_ref, acc):
    i = pl.program_id(0)
    dinv = lax.rsqrt(dg_ref[...] + 1.0)
    u0 = (s0_ref[...] + t0_ref[...]) * dinv
    u1 = (s1_ref[...] + t1_ref[...]) * dinv
    h0 = jnp.maximum(u0 + b3_ref[:, :_HALF], 0.0)
    h1 = jnp.maximum(u1 + b3_ref[:, _HALF:], 0.0)

    bvals = batch_ref[...].reshape(1, _PBLK)
    gids = lax.broadcasted_iota(jnp.int32, (_G, _PBLK), 0)
    mask = (bvals == gids).astype(jnp.float32)

    @pl.when(i == 0)
    def _():
        acc[...] = jnp.zeros((_G, _H), jnp.float32)

    acc[:, :_HALF] += jnp.dot(mask, h0, preferred_element_type=jnp.float32)
    acc[:, _HALF:] += jnp.dot(mask, h1, preferred_element_type=jnp.float32)

    @pl.when(i == pl.num_programs(0) - 1)
    def _():
        p = acc[...]
        z1 = jnp.maximum(
            jnp.dot(p, w1_ref[...], preferred_element_type=jnp.float32)
            + b1_ref[...], 0.0)
        z = (jnp.dot(z1, w2_ref[...], preferred_element_type=jnp.float32)
             + b2_ref[...])
        m = jnp.max(z, axis=1, keepdims=True)
        e = jnp.exp(z - m)
        lse = jnp.log(jnp.sum(e, axis=1, keepdims=True)) + m
        o_ref[...] = z - lse


def _head_call(ss, tts, dg, b3, batch3d, w1, b1, w2, b2):
    return pl.pallas_call(
        _head_body,
        grid=(_NP // _PBLK,),
        in_specs=[
            pl.BlockSpec((_PBLK, _HALF), lambda i: (i, 0)),
            pl.BlockSpec((_PBLK, _HALF), lambda i: (i, 0)),
            pl.BlockSpec((_PBLK, _HALF), lambda i: (i, 0)),
            pl.BlockSpec((_PBLK, _HALF), lambda i: (i, 0)),
            pl.BlockSpec((_PBLK, 1), lambda i: (i, 0)),
            pl.BlockSpec((1, _H), lambda i: (0, 0)),
            pl.BlockSpec((1, 1, _PBLK), lambda i: (i, 0, 0)),
            pl.BlockSpec((_H, _H), lambda i: (0, 0)),
            pl.BlockSpec((1, _H), lambda i: (0, 0)),
            pl.BlockSpec((_H, _C), lambda i: (0, 0)),
            pl.BlockSpec((1, _C), lambda i: (0, 0)),
        ],
        out_specs=pl.BlockSpec((_G, _C), lambda i: (0, 0)),
        out_shape=jax.ShapeDtypeStruct((_G, _C), jnp.float32),
        scratch_shapes=[pltpu.VMEM((_G, _H), jnp.float32)],
    )(*ss, *tts, dg, b3, batch3d, w1, b1, w2, b2)


def kernel(x, edge_index, batch, W1, b1, W2, b2, W3, b3,
           lin1_W, lin1_b, lin2_W, lin2_b):
    x_pad = jnp.pad(x, ((0, _NP - _N), (0, 0)))
    src = jnp.concatenate(
        [edge_index[0], jnp.zeros((_EP - _E,), jnp.int32)])
    dst_deg = jnp.concatenate(
        [edge_index[1], jnp.full((_EP - _E,), _DUMMY, jnp.int32)])
    dst_sent = jnp.concatenate(
        [edge_index[1], jnp.full((_EP - _E,), 1 << 20, jnp.int32)])
    dst2d = dst_deg.reshape(_EP // _CHUNK, _CHUNK)

    batch3d = jnp.pad(batch, (0, _NP - _N), constant_values=_G).reshape(
        _NP // _PBLK, 1, _PBLK)
    b1r = b1.reshape(1, _H)
    b2r = b2.reshape(1, _H)
    b3r = b3.reshape(1, _H)
    l1br = lin1_b.reshape(1, _H)
    l2br = lin2_b.reshape(1, _C)

    ps1, pd1 = _part_call(src, dst_sent)
    psrc = ps1.reshape(_PROWS, _CHUNK)
    pdst = pd1.reshape(_PSIZE_D // _CHUNK, _CHUNK)

    h0, h1 = _deg_call(dst2d)
    dg = (h0 + h1).reshape(_NP, 1)
    tts = _k1_call(x_pad, W1, dg)
    ss = _scat_call(tts, psrc, pdst)
    tts = _layer_call(ss, tts, dg, b1r, W2)
    ss = _scat_call(tts, psrc, pdst)
    tts = _layer_call(ss, tts, dg, b2r, W3)
    ss = _scat_call(tts, psrc, pdst)
    return _head_call(ss, tts, dg, b3r, batch3d,
                      lin1_W, l1br, lin2_W, l2br)
